# baseline (speedup 1.0000x reference)
import functools
import jax
import jax.numpy as jnp
from jax import lax
from jax.experimental import pallas as pl
from jax.experimental.pallas import tpu as pltpu
from jax.experimental.pallas import tpu_sc as plsc

_N = 64
_NQ = 9
_RB = 128
_NW = 32
_ROWS = 1024 * 512
_RPW = _ROWS // _NW
_NBLK = _RPW // _RB


def _sort_desc(v):
    return plsc.sort_key_val(v, v, descending=True)[0]


def _sort64(a, b, c, d):
    a = lax.sort(a)
    b = _sort_desc(b)
    c = lax.sort(c)
    d = _sort_desc(d)
    lo, hi = jnp.minimum(a, b), jnp.maximum(a, b)
    a2, b2 = lax.sort(lo), lax.sort(hi)
    lo, hi = jnp.minimum(c, d), jnp.maximum(c, d)
    c2, d2 = _sort_desc(hi), _sort_desc(lo)
    l0, l1 = jnp.minimum(a2, c2), jnp.minimum(b2, d2)
    h0, h1 = jnp.maximum(a2, c2), jnp.maximum(b2, d2)
    s0 = lax.sort(jnp.minimum(l0, l1))
    s1 = lax.sort(jnp.maximum(l0, l1))
    s2 = lax.sort(jnp.minimum(h0, h1))
    s3 = lax.sort(jnp.maximum(h0, h1))
    return s0, s1, s2, s3


def _make_kernel():
    mesh = plsc.VectorSubcoreMesh(core_axis_name="c", subcore_axis_name="s")

    @functools.partial(
        pl.kernel,
        mesh=mesh,
        out_type=jax.ShapeDtypeStruct((_ROWS, _NQ), jnp.float32),
        scratch_types=[
            pltpu.VMEM((_RB, _N), jnp.float32),
            pltpu.VMEM((_RB, _N), jnp.float32),
            pltpu.VMEM((_RB, _NQ), jnp.float32),
            pltpu.SemaphoreType.DMA,
            pltpu.SemaphoreType.DMA,
        ],
        compiler_params=pltpu.CompilerParams(
            needs_layout_passes=False, use_tc_tiling_on_sc=True),
    )
    def k(x_hbm, out_hbm, x_v0, x_v1, o_v, sem0, sem1):
        wid = lax.axis_index("s") * 2 + lax.axis_index("c")
        base_row = wid * _RPW
        lane = lax.iota(jnp.int32, 16)
        m0 = (lane == 6) | (lane == 13)
        m1 = (lane == 3) | (lane == 9)
        m2 = (lane == 0) | (lane == 6) | (lane == 12)
        m3 = (lane == 2) | (lane == 9)
        i0 = jnp.where(lane == 13, 1, 0)
        i1 = jnp.where(lane == 3, 2, 3)
        i2 = jnp.where(lane == 0, 4, jnp.where(lane == 6, 5, 6))
        i3 = jnp.where(lane == 2, 7, 8)

        def in_copy(blk, buf, sem):
            start = base_row + blk * _RB
            return pltpu.make_async_copy(
                x_hbm.at[pl.ds(start, _RB), :], buf, sem)

        def process(blk, buf):
            @plsc.parallel_loop(0, _RB, 1, unroll=4)
            def row_body(r):
                a = buf[r, pl.ds(0, 16)]
                b = buf[r, pl.ds(16, 16)]
                c = buf[r, pl.ds(32, 16)]
                d = buf[r, pl.ds(48, 16)]
                s0, s1, s2, s3 = _sort64(a, b, c, d)
                rv = i0 * 0 + r
                plsc.store_scatter(o_v, [rv, i0], s0, mask=m0)
                plsc.store_scatter(o_v, [rv, i1], s1, mask=m1)
                plsc.store_scatter(o_v, [rv, i2], s2, mask=m2)
                plsc.store_scatter(o_v, [rv, i3], s3, mask=m3)

            start = base_row + blk * _RB
            pltpu.sync_copy(o_v, out_hbm.at[pl.ds(start, _RB), :])

        in_copy(0, x_v0, sem0).start()
        in_copy(1, x_v1, sem1).start()

        def pair_body(p, carry):
            blk = 2 * p
            in_copy(blk, x_v0, sem0).wait()
            process(blk, x_v0)

            @pl.when(blk + 2 < _NBLK)
            def _():
                in_copy(blk + 2, x_v0, sem0).start()

            in_copy(blk + 1, x_v1, sem1).wait()
            process(blk + 1, x_v1)

            @pl.when(blk + 3 < _NBLK)
            def _():
                in_copy(blk + 3, x_v1, sem1).start()

            return carry

        lax.fori_loop(0, _NBLK // 2, pair_body, 0)

    return k


def kernel(x):
    b, t, n = x.shape
    out = _make_kernel()(x.reshape(b * t, n))
    return out.reshape(b, t, _NQ)


# speedup vs baseline: 1.0329x; 1.0329x over previous
import functools
import jax
import jax.numpy as jnp
from jax import lax
from jax.experimental import pallas as pl
from jax.experimental.pallas import tpu as pltpu
from jax.experimental.pallas import tpu_sc as plsc

_N = 64
_NQ = 9
_RB = 256
_NW = 32
_ROWS = 1024 * 512
_RPW = _ROWS // _NW
_NBLK = _RPW // _RB


def _sort_desc(v):
    return plsc.sort_key_val(v, v, descending=True)[0]


def _sort64(a, b, c, d):
    a = lax.sort(a)
    b = _sort_desc(b)
    c = lax.sort(c)
    d = _sort_desc(d)
    lo, hi = jnp.minimum(a, b), jnp.maximum(a, b)
    a2, b2 = lax.sort(lo), lax.sort(hi)
    lo, hi = jnp.minimum(c, d), jnp.maximum(c, d)
    c2, d2 = _sort_desc(hi), _sort_desc(lo)
    l0, l1 = jnp.minimum(a2, c2), jnp.minimum(b2, d2)
    h0, h1 = jnp.maximum(a2, c2), jnp.maximum(b2, d2)
    s0 = lax.sort(jnp.minimum(l0, l1))
    s1 = lax.sort(jnp.maximum(l0, l1))
    s2 = lax.sort(jnp.minimum(h0, h1))
    s3 = lax.sort(jnp.maximum(h0, h1))
    return s0, s1, s2, s3


def _make_kernel():
    mesh = plsc.VectorSubcoreMesh(core_axis_name="c", subcore_axis_name="s")

    @functools.partial(
        pl.kernel,
        mesh=mesh,
        out_type=jax.ShapeDtypeStruct((_ROWS, _NQ), jnp.float32),
        scratch_types=[
            pltpu.VMEM((_RB, _N), jnp.float32),
            pltpu.VMEM((_RB, _N), jnp.float32),
            pltpu.VMEM((_RB, _NQ), jnp.float32),
            pltpu.SemaphoreType.DMA,
            pltpu.SemaphoreType.DMA,
        ],
        compiler_params=pltpu.CompilerParams(
            needs_layout_passes=False, use_tc_tiling_on_sc=True),
    )
    def k(x_hbm, out_hbm, x_v0, x_v1, o_v, sem0, sem1):
        wid = lax.axis_index("s") * 2 + lax.axis_index("c")
        base_row = wid * _RPW
        lane = lax.iota(jnp.int32, 16)
        m0 = (lane == 6) | (lane == 13)
        m1 = (lane == 3) | (lane == 9)
        m2 = (lane == 0) | (lane == 6) | (lane == 12)
        m3 = (lane == 2) | (lane == 9)
        i0 = jnp.where(lane == 13, 1, 0)
        i1 = jnp.where(lane == 3, 2, 3)
        i2 = jnp.where(lane == 0, 4, jnp.where(lane == 6, 5, 6))
        i3 = jnp.where(lane == 2, 7, 8)

        def in_copy(blk, buf, sem):
            start = base_row + blk * _RB
            return pltpu.make_async_copy(
                x_hbm.at[pl.ds(start, _RB), :], buf, sem)

        def process(blk, buf):
            @plsc.parallel_loop(0, _RB, 1, unroll=4)
            def row_body(r):
                a = buf[r, pl.ds(0, 16)]
                b = buf[r, pl.ds(16, 16)]
                c = buf[r, pl.ds(32, 16)]
                d = buf[r, pl.ds(48, 16)]
                s0, s1, s2, s3 = _sort64(a, b, c, d)
                rv = i0 * 0 + r
                plsc.store_scatter(o_v, [rv, i0], s0, mask=m0)
                plsc.store_scatter(o_v, [rv, i1], s1, mask=m1)
                plsc.store_scatter(o_v, [rv, i2], s2, mask=m2)
                plsc.store_scatter(o_v, [rv, i3], s3, mask=m3)

            start = base_row + blk * _RB
            pltpu.sync_copy(o_v, out_hbm.at[pl.ds(start, _RB), :])

        in_copy(0, x_v0, sem0).start()
        in_copy(1, x_v1, sem1).start()

        def pair_body(p, carry):
            blk = 2 * p
            in_copy(blk, x_v0, sem0).wait()
            process(blk, x_v0)

            @pl.when(blk + 2 < _NBLK)
            def _():
                in_copy(blk + 2, x_v0, sem0).start()

            in_copy(blk + 1, x_v1, sem1).wait()
            process(blk + 1, x_v1)

            @pl.when(blk + 3 < _NBLK)
            def _():
                in_copy(blk + 3, x_v1, sem1).start()

            return carry

        lax.fori_loop(0, _NBLK // 2, pair_body, 0)

    return k


def kernel(x):
    b, t, n = x.shape
    out = _make_kernel()(x.reshape(b * t, n))
    return out.reshape(b, t, _NQ)
